# baseline (speedup 1.0000x reference)
import numpy as np

import jax
import jax.numpy as jnp
from jax import lax
from jax.experimental import pallas as pl
from jax.experimental.pallas import tpu as pltpu
from jax.experimental.pallas import tpu_sc as plsc

ROWS = 128
N = 32768
L = 16
NC = 2
NS = 16
NW = NC * NS
RPW = ROWS // NW

NPASS = 4
NB = 256
C = 8
CV = N // (C * L)
CELEM_SHIFT = 12

MININT = np.int32(-2147483648)


def _to_key(v):
    m = v >> 31
    return v ^ (m | MININT)


def _from_key(k):
    m = k >> 31
    return k ^ (~m | MININT)


def _digit(k, p):
    return lax.shift_right_logical(k, jnp.int32(8 * p)) & jnp.int32(NB - 1)


def _zero(ref, n):
    zeros = jnp.zeros((L,), jnp.int32)

    def body(i, c):
        ref[pl.ds(i * L, L)] = zeros
        return c

    lax.fori_loop(0, n // L, body, 0)


def _body(x_hbm, out_hbm, buf_a, buf_b, cnt_a, cnt_b, gtmp, *ptrs):
    wid = lax.axis_index("s") * NC + lax.axis_index("c")

    def row_body(r, c0):
        row = wid * RPW + r

        pltpu.sync_copy(x_hbm.at[row], buf_a)
        _zero(cnt_a, C * NB)

        def sweep0(v, c):
            for u in range(C):
                sl = pl.ds((u * CV + v) * L, L)
                k = _to_key(buf_a[sl])
                buf_a[sl] = k
                d = _digit(k, 0)
                cnt, lastm = plsc.scan_count(d)
                plsc.addupdate_scatter(cnt_a, [d + jnp.int32(u * NB)], cnt,
                                       mask=lastm)
            return c

        lax.fori_loop(0, CV, sweep0, 0)

        def build_ptrs(cnt):
            def totals(i, c):
                ds = pl.ds(i * L, L)
                tot = cnt[pl.ds(i * L, L)]
                for u in range(1, C):
                    tot = tot + cnt[pl.ds(u * NB + i * L, L)]
                gtmp[ds] = tot
                return c

            lax.fori_loop(0, NB // L, totals, 0)

            def excl(i, carry):
                ds = pl.ds(i * L, L)
                h = gtmp[ds]
                inc = plsc.cumsum(h)
                gtmp[ds] = inc - h + carry
                return carry + jnp.sum(h)

            lax.fori_loop(0, NB // L, excl, jnp.int32(0))

            def bases(i, c):
                ds = pl.ds(i * L, L)
                run = gtmp[ds]
                for u in range(C):
                    ptrs[u][ds] = run
                    run = run + cnt[pl.ds(u * NB + i * L, L)]
                return c

            lax.fori_loop(0, NB // L, bases, 0)

        def permute(p, src, dst, cnt_cur, cnt_next):
            build_ptrs(cnt_cur)
            if cnt_next is not None:
                _zero(cnt_next, C * NB)

            def sweep(v, c):
                for u in range(C):
                    sl = pl.ds((u * CV + v) * L, L)
                    k = src[sl]
                    d = _digit(k, p)
                    cnt, lastm = plsc.scan_count(d)
                    base = plsc.load_gather(ptrs[u], [d])
                    off = base + cnt - 1
                    val = k if p < NPASS - 1 else _from_key(k)
                    plsc.store_scatter(dst, [off], val)
                    plsc.store_scatter(ptrs[u], [d], base + cnt, mask=lastm)
                    if cnt_next is not None:
                        d2 = _digit(k, p + 1)
                        idx2 = lax.shift_left(
                            lax.shift_right_logical(off, jnp.int32(CELEM_SHIFT)),
                            jnp.int32(8)) | d2
                        cnt2, last2 = plsc.scan_count(idx2)
                        plsc.addupdate_scatter(cnt_next, [idx2], cnt2,
                                               mask=last2)
                return c

            lax.fori_loop(0, CV, sweep, 0)

        permute(0, buf_a, buf_b, cnt_a, cnt_b)
        permute(1, buf_b, buf_a, cnt_b, cnt_a)
        permute(2, buf_a, buf_b, cnt_a, cnt_b)
        permute(3, buf_b, buf_a, cnt_b, None)

        pltpu.sync_copy(buf_a, out_hbm.at[row])
        return c0

    lax.fori_loop(0, RPW, row_body, 0)


@jax.jit
def kernel(x):
    xi = lax.bitcast_convert_type(x, jnp.int32)
    mesh = plsc.VectorSubcoreMesh(core_axis_name="c", subcore_axis_name="s")
    sort_rows = pl.kernel(
        _body,
        out_type=jax.ShapeDtypeStruct((ROWS, N), jnp.int32),
        mesh=mesh,
        compiler_params=pltpu.CompilerParams(needs_layout_passes=False),
        scratch_types=[
            pltpu.VMEM((N,), jnp.int32),
            pltpu.VMEM((N,), jnp.int32),
            pltpu.VMEM((C * NB,), jnp.int32),
            pltpu.VMEM((C * NB,), jnp.int32),
            pltpu.VMEM((NB,), jnp.int32),
        ] + [pltpu.VMEM((NB,), jnp.int32) for _ in range(C)],
    )
    oi = sort_rows(xi)
    return lax.bitcast_convert_type(oi, jnp.float32)


# speedup vs baseline: 2.7048x; 2.7048x over previous
import numpy as np

import jax
import jax.numpy as jnp
from jax import lax
from jax.experimental import pallas as pl
from jax.experimental.pallas import tpu as pltpu
from jax.experimental.pallas import tpu_sc as plsc

ROWS = 128
N = 32768
L = 16
NV = N // L
NC = 2
NS = 16
NW = NC * NS
RPW = ROWS // NW

NB = 2048
SHIFTS = (0, 11, 22)
MASKS = (2047, 2047, 1023)
NBINS = (2048, 2048, 1024)

MININT = np.int32(-2147483648)


def _to_key(v):
    m = v >> 31
    return v ^ (m | MININT)


def _from_key(k):
    m = k >> 31
    return k ^ (~m | MININT)


def _digit(k, p):
    return lax.shift_right_logical(k, jnp.int32(SHIFTS[p])) & jnp.int32(MASKS[p])


def _zero_hist(hist, nbins):
    zeros = jnp.zeros((L,), jnp.int32)

    def body(i, c):
        hist[pl.ds(i * L, L)] = zeros
        return c

    lax.fori_loop(0, nbins // L, body, 0)


def _exclusive_scan(hist, nbins):
    def body(i, carry):
        h = hist[pl.ds(i * L, L)]
        inc = plsc.cumsum(h)
        hist[pl.ds(i * L, L)] = inc - h + carry
        return carry + jnp.sum(h)

    lax.fori_loop(0, nbins // L, body, jnp.int32(0))


UNROLL = 4


def _body(x_hbm, out_hbm, buf_a, buf_b, hist_0, hist_1, hist_2):
    wid = lax.axis_index("s") * NC + lax.axis_index("c")
    hists = (hist_0, hist_1, hist_2)

    def row_body(r, c0):
        row = wid * RPW + r

        pltpu.sync_copy(x_hbm.at[row], buf_a)
        for p in range(3):
            _zero_hist(hists[p], NBINS[p])

        def sweep0(i, c):
            ks = []
            for u in range(UNROLL):
                v = buf_a[pl.ds((i * UNROLL + u) * L, L)]
                k = _to_key(v)
                buf_a[pl.ds((i * UNROLL + u) * L, L)] = k
                ks.append(k)
            digs = [[_digit(k, p) for k in ks] for p in range(3)]
            for p in range(3):
                scans = [plsc.scan_count(d) for d in digs[p]]
                for u in range(UNROLL):
                    cnt, lastm = scans[u]
                    plsc.addupdate_scatter(hists[p], [digs[p][u]], cnt,
                                           mask=lastm)
            return c

        lax.fori_loop(0, NV // UNROLL, sweep0, 0)

        def permute(p, src, dst):
            hist = hists[p]
            _exclusive_scan(hist, NBINS[p])

            def sweep(i, c):
                ks = [src[pl.ds((i * UNROLL + u) * L, L)]
                      for u in range(UNROLL)]
                digs = [_digit(k, p) for k in ks]
                scans = [plsc.scan_count(d) for d in digs]
                vals = ks if p < 2 else [_from_key(k) for k in ks]
                for u in range(UNROLL):
                    cnt, lastm = scans[u]
                    d = digs[u]
                    base = plsc.load_gather(hist, [d])
                    off = base + cnt - 1
                    plsc.store_scatter(dst, [off], vals[u])
                    plsc.store_scatter(hist, [d], base + cnt, mask=lastm)
                return c

            lax.fori_loop(0, NV // UNROLL, sweep, 0)

        permute(0, buf_a, buf_b)
        permute(1, buf_b, buf_a)
        permute(2, buf_a, buf_b)

        pltpu.sync_copy(buf_b, out_hbm.at[row])
        return c0

    lax.fori_loop(0, RPW, row_body, 0)


@jax.jit
def kernel(x):
    xi = lax.bitcast_convert_type(x, jnp.int32)
    mesh = plsc.VectorSubcoreMesh(core_axis_name="c", subcore_axis_name="s")
    sort_rows = pl.kernel(
        _body,
        out_type=jax.ShapeDtypeStruct((ROWS, N), jnp.int32),
        mesh=mesh,
        compiler_params=pltpu.CompilerParams(needs_layout_passes=False),
        scratch_types=[
            pltpu.VMEM((N,), jnp.int32),
            pltpu.VMEM((N,), jnp.int32),
            pltpu.VMEM((NBINS[0],), jnp.int32),
            pltpu.VMEM((NBINS[1],), jnp.int32),
            pltpu.VMEM((NBINS[2],), jnp.int32),
        ],
    )
    oi = sort_rows(xi)
    return lax.bitcast_convert_type(oi, jnp.float32)


# speedup vs baseline: 3.0961x; 1.1447x over previous
import numpy as np

import jax
import jax.numpy as jnp
from jax import lax
from jax.experimental import pallas as pl
from jax.experimental.pallas import tpu as pltpu
from jax.experimental.pallas import tpu_sc as plsc

ROWS = 128
N = 32768
L = 16
NV = N // L
NC = 2
NS = 16
NW = NC * NS
RPW = ROWS // NW

NB = 2048
SHIFTS = (0, 11, 22)
MASKS = (2047, 2047, 1023)
NBINS = (2048, 2048, 1024)

MININT = np.int32(-2147483648)


def _to_key(v):
    m = v >> 31
    return v ^ (m | MININT)


def _from_key(k):
    m = k >> 31
    return k ^ (~m | MININT)


def _digit(k, p):
    return lax.shift_right_logical(k, jnp.int32(SHIFTS[p])) & jnp.int32(MASKS[p])


def _zero_hist(hist, nbins):
    zeros = jnp.zeros((L,), jnp.int32)

    def body(i, c):
        hist[pl.ds(i * L, L)] = zeros
        return c

    lax.fori_loop(0, nbins // L, body, 0)


def _exclusive_scan(hist, nbins):
    def body(i, carry):
        h = hist[pl.ds(i * L, L)]
        inc = plsc.cumsum(h)
        hist[pl.ds(i * L, L)] = inc - h + carry
        return carry + jnp.sum(h)

    lax.fori_loop(0, nbins // L, body, jnp.int32(0))


UNROLL = 8


def _body(x_hbm, out_hbm, buf_a, buf_b, hist_0, hist_1, hist_2):
    wid = lax.axis_index("s") * NC + lax.axis_index("c")
    hists = (hist_0, hist_1, hist_2)

    def row_body(r, c0):
        row = wid * RPW + r

        pltpu.sync_copy(x_hbm.at[row], buf_a)
        for p in range(3):
            _zero_hist(hists[p], NBINS[p])

        def sweep0(i, c):
            ks = []
            for u in range(UNROLL):
                v = buf_a[pl.ds((i * UNROLL + u) * L, L)]
                k = _to_key(v)
                buf_a[pl.ds((i * UNROLL + u) * L, L)] = k
                ks.append(k)
            digs = [[_digit(k, p) for k in ks] for p in range(3)]
            for p in range(3):
                scans = [plsc.scan_count(d) for d in digs[p]]
                for u in range(UNROLL):
                    cnt, lastm = scans[u]
                    plsc.addupdate_scatter(hists[p], [digs[p][u]], cnt,
                                           mask=lastm)
            return c

        lax.fori_loop(0, NV // UNROLL, sweep0, 0)

        def permute(p, src, dst):
            hist = hists[p]
            _exclusive_scan(hist, NBINS[p])

            def sweep(i, c):
                ks = [src[pl.ds((i * UNROLL + u) * L, L)]
                      for u in range(UNROLL)]
                digs = [_digit(k, p) for k in ks]
                scans = [plsc.scan_count(d) for d in digs]
                vals = ks if p < 2 else [_from_key(k) for k in ks]
                for u in range(UNROLL):
                    cnt, lastm = scans[u]
                    d = digs[u]
                    base = plsc.load_gather(hist, [d])
                    off = base + cnt - 1
                    plsc.store_scatter(dst, [off], vals[u])
                    plsc.store_scatter(hist, [d], base + cnt, mask=lastm)
                return c

            lax.fori_loop(0, NV // UNROLL, sweep, 0)

        permute(0, buf_a, buf_b)
        permute(1, buf_b, buf_a)
        permute(2, buf_a, buf_b)

        pltpu.sync_copy(buf_b, out_hbm.at[row])
        return c0

    lax.fori_loop(0, RPW, row_body, 0)


@jax.jit
def kernel(x):
    xi = lax.bitcast_convert_type(x, jnp.int32)
    mesh = plsc.VectorSubcoreMesh(core_axis_name="c", subcore_axis_name="s")
    sort_rows = pl.kernel(
        _body,
        out_type=jax.ShapeDtypeStruct((ROWS, N), jnp.int32),
        mesh=mesh,
        compiler_params=pltpu.CompilerParams(needs_layout_passes=False),
        scratch_types=[
            pltpu.VMEM((N,), jnp.int32),
            pltpu.VMEM((N,), jnp.int32),
            pltpu.VMEM((NBINS[0],), jnp.int32),
            pltpu.VMEM((NBINS[1],), jnp.int32),
            pltpu.VMEM((NBINS[2],), jnp.int32),
        ],
    )
    oi = sort_rows(xi)
    return lax.bitcast_convert_type(oi, jnp.float32)


# speedup vs baseline: 3.3169x; 1.0713x over previous
import numpy as np

import jax
import jax.numpy as jnp
from jax import lax
from jax.experimental import pallas as pl
from jax.experimental.pallas import tpu as pltpu
from jax.experimental.pallas import tpu_sc as plsc

ROWS = 128
N = 32768
L = 16
NV = N // L
NC = 2
NS = 16
NW = NC * NS
RPW = ROWS // NW

NB = 2048
SHIFTS = (0, 11, 22)
MASKS = (2047, 2047, 1023)
NBINS = (2048, 2048, 1024)

MININT = np.int32(-2147483648)


def _to_key(v):
    m = v >> 31
    return v ^ (m | MININT)


def _from_key(k):
    m = k >> 31
    return k ^ (~m | MININT)


def _digit(k, p):
    return lax.shift_right_logical(k, jnp.int32(SHIFTS[p])) & jnp.int32(MASKS[p])


def _zero_hist(hist, nbins):
    zeros = jnp.zeros((L,), jnp.int32)

    def body(i, c):
        hist[pl.ds(i * L, L)] = zeros
        return c

    lax.fori_loop(0, nbins // L, body, 0)


def _exclusive_scan(hist, nbins):
    def body(i, carry):
        h = hist[pl.ds(i * L, L)]
        inc = plsc.cumsum(h)
        hist[pl.ds(i * L, L)] = inc - h + carry
        return carry + jnp.sum(h)

    lax.fori_loop(0, nbins // L, body, jnp.int32(0))


UNROLL = 16


def _body(x_hbm, out_hbm, buf_a, buf_b, hist_0, hist_1, hist_2):
    wid = lax.axis_index("s") * NC + lax.axis_index("c")
    hists = (hist_0, hist_1, hist_2)

    def row_body(r, c0):
        row = wid * RPW + r

        pltpu.sync_copy(x_hbm.at[row], buf_a)
        for p in range(3):
            _zero_hist(hists[p], NBINS[p])

        def sweep0(i, c):
            ks = []
            for u in range(UNROLL):
                v = buf_a[pl.ds((i * UNROLL + u) * L, L)]
                k = _to_key(v)
                buf_a[pl.ds((i * UNROLL + u) * L, L)] = k
                ks.append(k)
            digs = [[_digit(k, p) for k in ks] for p in range(3)]
            for p in range(3):
                scans = [plsc.scan_count(d) for d in digs[p]]
                for u in range(UNROLL):
                    cnt, lastm = scans[u]
                    plsc.addupdate_scatter(hists[p], [digs[p][u]], cnt,
                                           mask=lastm)
            return c

        lax.fori_loop(0, NV // UNROLL, sweep0, 0)

        def permute(p, src, dst):
            hist = hists[p]
            _exclusive_scan(hist, NBINS[p])

            def sweep(i, c):
                ks = [src[pl.ds((i * UNROLL + u) * L, L)]
                      for u in range(UNROLL)]
                digs = [_digit(k, p) for k in ks]
                scans = [plsc.scan_count(d) for d in digs]
                vals = ks if p < 2 else [_from_key(k) for k in ks]
                for u in range(UNROLL):
                    cnt, lastm = scans[u]
                    d = digs[u]
                    base = plsc.load_gather(hist, [d])
                    off = base + cnt - 1
                    plsc.store_scatter(dst, [off], vals[u])
                    plsc.store_scatter(hist, [d], base + cnt, mask=lastm)
                return c

            lax.fori_loop(0, NV // UNROLL, sweep, 0)

        permute(0, buf_a, buf_b)
        permute(1, buf_b, buf_a)
        permute(2, buf_a, buf_b)

        pltpu.sync_copy(buf_b, out_hbm.at[row])
        return c0

    lax.fori_loop(0, RPW, row_body, 0)


@jax.jit
def kernel(x):
    xi = lax.bitcast_convert_type(x, jnp.int32)
    mesh = plsc.VectorSubcoreMesh(core_axis_name="c", subcore_axis_name="s")
    sort_rows = pl.kernel(
        _body,
        out_type=jax.ShapeDtypeStruct((ROWS, N), jnp.int32),
        mesh=mesh,
        compiler_params=pltpu.CompilerParams(needs_layout_passes=False),
        scratch_types=[
            pltpu.VMEM((N,), jnp.int32),
            pltpu.VMEM((N,), jnp.int32),
            pltpu.VMEM((NBINS[0],), jnp.int32),
            pltpu.VMEM((NBINS[1],), jnp.int32),
            pltpu.VMEM((NBINS[2],), jnp.int32),
        ],
    )
    oi = sort_rows(xi)
    return lax.bitcast_convert_type(oi, jnp.float32)
